# baseline (speedup 1.0000x reference)
import dataclasses
import functools

import jax
import jax.numpy as jnp
from jax import lax
from jax.experimental import pallas as pl
from jax.experimental.pallas import tpu as pltpu
from jax.experimental.pallas import tpu_sc as plsc

NC = 2
NS = 16
L = 16
NW = NC * NS

BLK = 128
BLK_PER_CHUNK = 4
CHUNK_E = BLK * BLK_PER_CHUNK
CPS = 4
SUP_E = CPS * CHUNK_E
SUP_ROWS = 3 * BLK_PER_CHUNK * CPS


def _sc_weighted_segsum(h_pad, epacked, n_nodes, nsuper):
    mesh = plsc.VectorSubcoreMesh(core_axis_name="c", subcore_axis_name="s")
    rows_per_sub = n_nodes // NS
    zrows = rows_per_sub // 16
    assert rows_per_sub % zrows == 0 and zrows <= CHUNK_E
    assert nsuper % 2 == 0
    bpc = BLK_PER_CHUNK

    cp = pltpu.CompilerParams()
    if "needs_layout_passes" in pltpu.CompilerParams.__dataclass_fields__:
        cp = dataclasses.replace(cp, needs_layout_passes=False)
    if "use_tc_tiling_on_sc" in pltpu.CompilerParams.__dataclass_fields__:
        cp = dataclasses.replace(cp, use_tc_tiling_on_sc=False)

    @functools.partial(
        pl.kernel,
        mesh=mesh,
        compiler_params=cp,
        out_type=jax.ShapeDtypeStruct((NC, n_nodes, L), jnp.float32),
        scratch_types=[
            pltpu.VMEM((SUP_ROWS, BLK), jnp.int32),
            pltpu.VMEM((SUP_ROWS, BLK), jnp.int32),
            pltpu.VMEM((CHUNK_E, L), jnp.float32),
            pltpu.VMEM((CHUNK_E, L), jnp.float32),
            pltpu.VMEM_SHARED((n_nodes, L), jnp.float32),
            pltpu.SemaphoreType.DMA,
            pltpu.SemaphoreType.DMA,
            pltpu.SemaphoreType.DMA,
            pltpu.SemaphoreType.DMA,
            pltpu.SemaphoreType.DMA,
            pltpu.SemaphoreType.DMA,
        ],
    )
    def seg_kernel(h_hbm, e_hbm, out_hbm,
                   ebuf0, ebuf1, rows0, rows1, acc,
                   esem0, esem1, gsem0, gsem1, ssem0, ssem1):
        c = lax.axis_index("c")
        s = lax.axis_index("s")
        wid = s * NC + c
        rowsb = (rows0, rows1)
        gsem = (gsem0, gsem1)
        ssem = (ssem0, ssem1)

        @pl.loop(0, zrows)
        def _zfill(i):
            rows0[i, :] = jnp.zeros((L,), jnp.float32)

        zcopies = [
            pltpu.async_copy(
                rows0.at[pl.ds(0, zrows)],
                acc.at[pl.ds(s * rows_per_sub + i * zrows, zrows)], esem0)
            for i in range(rows_per_sub // zrows)
        ]
        for z in zcopies:
            z.wait()

        plsc.subcore_barrier()

        sup0 = wid * nsuper

        def fire_gathers(ebuf, q, rows, sem):
            return [pltpu.async_copy(h_hbm.at[ebuf.at[12 * q + j]],
                                     rows.at[pl.ds(j * BLK, BLK)], sem)
                    for j in range(bpc)]

        def scale_rows(ebuf, q, rows):
            for j in range(bpc):
                wrow = 12 * q + 2 * bpc + j

                @pl.loop(0, BLK, step=8)
                def _scale(e, j=j, wrow=wrow):
                    for k in range(8):
                        wb_i = plsc.load_gather(
                            ebuf, [jnp.full((L,), wrow, jnp.int32),
                                   jnp.full((L,), e + k, jnp.int32)])
                        wb = plsc.bitcast(wb_i, jnp.float32)
                        idx = j * BLK + e + k
                        rows[idx, :] = rows[idx, :] * wb

        def fire_scatters(ebuf, q, rows, sem):
            return [pltpu.async_copy(rows.at[pl.ds(j * BLK, BLK)],
                                     acc.at[ebuf.at[12 * q + bpc + j]], sem,
                                     add=True)
                    for j in range(bpc)]

        def process4(ebuf):
            for qq in range(0, CPS, 2):
                g0 = fire_gathers(ebuf, qq, rows0, gsem0)
                g1 = fire_gathers(ebuf, qq + 1, rows1, gsem1)
                for h in g0:
                    h.wait()
                scale_rows(ebuf, qq, rows0)
                s0 = fire_scatters(ebuf, qq, rows0, ssem0)
                for h in g1:
                    h.wait()
                scale_rows(ebuf, qq + 1, rows1)
                s1 = fire_scatters(ebuf, qq + 1, rows1, ssem1)
                for h in s0 + s1:
                    h.wait()
            return []

        pltpu.async_copy(e_hbm.at[sup0], ebuf0, esem0)

        @pl.loop(0, nsuper, step=2)
        def _pair(t):
            pltpu.make_async_copy(e_hbm.at[sup0 + t], ebuf0, esem0).wait()
            pltpu.async_copy(e_hbm.at[sup0 + t + 1], ebuf1, esem1)
            sa = process4(ebuf0)
            pltpu.make_async_copy(e_hbm.at[sup0 + t + 1], ebuf1, esem1).wait()
            for h in sa:
                h.wait()

            @pl.when(t + 2 < nsuper)
            def _next():
                pltpu.async_copy(e_hbm.at[sup0 + t + 2], ebuf0, esem0)

            sb = process4(ebuf1)
            for h in sb:
                h.wait()

        plsc.subcore_barrier()

        pltpu.sync_copy(acc.at[pl.ds(s * rows_per_sub, rows_per_sub)],
                        out_hbm.at[c].at[pl.ds(s * rows_per_sub, rows_per_sub)])

    return seg_kernel(h_pad, epacked)


def _norm_sq_max(hcat, n_nodes):
    br = 3128
    steps = n_nodes // br
    assert n_nodes % br == 0

    def body(h_ref, o_ref):
        i = pl.program_id(0)
        x = h_ref[...]
        m = jnp.max(jnp.sum(x * x, axis=1))

        @pl.when(i == 0)
        def _init():
            o_ref[0, 0] = m

        @pl.when(i > 0)
        def _acc():
            o_ref[0, 0] = jnp.maximum(o_ref[0, 0], m)

    return pl.pallas_call(
        body,
        grid=(steps,),
        in_specs=[pl.BlockSpec((br, L), lambda i: (i, 0))],
        out_specs=pl.BlockSpec(memory_space=pltpu.SMEM),
        out_shape=jax.ShapeDtypeStruct((1, 1), jnp.float32),
    )(hcat)


def _update_layer(h8, icat8, p08, p18, maxss, dt, wbd, btile,
                  scale_h, scale_agg, n8):
    def body(ms_ref, dt_ref, h_ref, i_ref, p0_ref, p1_ref, w_ref, b_ref, o_ref):
        s = lax.rsqrt(ms_ref[0, 0])
        dtv = dt_ref[0]
        h = h_ref[...]
        hs = h * s if scale_h else h
        agg = p0_ref[...] + p1_ref[...]
        a = jnp.dot(agg, w_ref[...], preferred_element_type=jnp.float32,
                    precision=lax.Precision.HIGHEST)
        if scale_agg:
            a = a * s
        t = jnp.tanh(a + b_ref[...])
        init_s = i_ref[...] * s
        o_ref[...] = hs + dtv * (t - hs + init_s)

    br = 3128
    assert n8 % br == 0
    return pl.pallas_call(
        body,
        grid=(n8 // br,),
        in_specs=[
            pl.BlockSpec(memory_space=pltpu.SMEM),
            pl.BlockSpec(memory_space=pltpu.SMEM),
            pl.BlockSpec((br, 128), lambda i: (i, 0)),
            pl.BlockSpec((br, 128), lambda i: (i, 0)),
            pl.BlockSpec((br, 128), lambda i: (i, 0)),
            pl.BlockSpec((br, 128), lambda i: (i, 0)),
            pl.BlockSpec((128, 128), lambda i: (0, 0)),
            pl.BlockSpec((1, 128), lambda i: (0, 0)),
        ],
        out_specs=pl.BlockSpec((br, 128), lambda i: (i, 0)),
        out_shape=jax.ShapeDtypeStruct((n8, 128), jnp.float32),
    )(maxss, dt, h8, icat8, p08, p18, wbd, btile)


def kernel(edge_index, edge_weight, dt, xu, xi, static_u, static_i,
           W0, b0, W1, b1):
    n_users, d = xu.shape
    n_items = xi.shape[0]
    n = n_users + n_items
    e = edge_weight.shape[0]
    npad = -(-n // 128) * 128
    n8 = npad * L // 128

    hcat = jnp.zeros((npad, L), jnp.float32)
    hcat = hcat.at[:n_users, :d].set(xu).at[n_users:n, :d].set(xi)
    icat = jnp.zeros((npad, L), jnp.float32)
    icat = icat.at[:n_users, :d].set(static_u).at[n_users:n, :d].set(static_i)

    nsuper = -(-e // (NW * SUP_E))
    nsuper += nsuper % 2
    e_pad = NW * SUP_E * nsuper
    pad = e_pad - e
    nsup_total = e_pad // SUP_E
    src_p = jnp.concatenate(
        [edge_index[0], jnp.zeros((pad,), jnp.int32)]
    ).reshape(nsup_total, CPS, BLK_PER_CHUNK, BLK)
    dst_p = jnp.concatenate(
        [edge_index[1], jnp.zeros((pad,), jnp.int32)]
    ).reshape(nsup_total, CPS, BLK_PER_CHUNK, BLK)
    w_p = jax.lax.bitcast_convert_type(
        jnp.concatenate([edge_weight, jnp.zeros((pad,), jnp.float32)]),
        jnp.int32).reshape(nsup_total, CPS, BLK_PER_CHUNK, BLK)
    epacked = jnp.concatenate([src_p, dst_p, w_p], axis=2).reshape(
        nsup_total, SUP_ROWS, BLK)

    def bdiag(w, b):
        wp = jnp.zeros((L, L), jnp.float32).at[:d, :d].set(w)
        bp = jnp.zeros((L,), jnp.float32).at[:d].set(b)
        return jnp.kron(jnp.eye(8, dtype=jnp.float32), wp), jnp.tile(bp, 8)[None, :]

    wbd0, bt0 = bdiag(W0, b0)
    wbd1, bt1 = bdiag(W1, b1)

    maxss = _norm_sq_max(hcat, npad)

    parts1 = _sc_weighted_segsum(hcat, epacked, npad, nsuper)
    p1a = parts1[0].reshape(n8, 128)
    p1b = parts1[1].reshape(n8, 128)
    h1_8 = _update_layer(hcat.reshape(n8, 128), icat.reshape(n8, 128),
                         p1a, p1b, maxss, dt, wbd0, bt0,
                         scale_h=True, scale_agg=True, n8=n8)

    h1 = h1_8.reshape(npad, L)
    parts2 = _sc_weighted_segsum(h1, epacked, npad, nsuper)
    p2a = parts2[0].reshape(n8, 128)
    p2b = parts2[1].reshape(n8, 128)
    h2_8 = _update_layer(h1_8, icat.reshape(n8, 128),
                         p2a, p2b, maxss, dt, wbd1, bt1,
                         scale_h=False, scale_agg=False, n8=n8)

    h2 = h2_8.reshape(npad, L)
    yu = h2[:n_users, :d]
    yi = h2[n_users:n, :d]
    return (yu, yi)


# speedup vs baseline: 1.9054x; 1.9054x over previous
import dataclasses
import functools

import jax
import jax.numpy as jnp
from jax import lax
from jax.experimental import pallas as pl
from jax.experimental.pallas import tpu as pltpu
from jax.experimental.pallas import tpu_sc as plsc

NC = 2
NS = 16
L = 16
NW = NC * NS

BLK = 128
BLK_PER_CHUNK = 4
CHUNK_E = BLK * BLK_PER_CHUNK
CPS = 2
SUP_E = CPS * CHUNK_E
SUP_ROWS = 3 * BLK_PER_CHUNK * CPS


def _sc_weighted_segsum(h_pad, epacked, n_nodes, nsuper):
    mesh = plsc.VectorSubcoreMesh(core_axis_name="c", subcore_axis_name="s")
    rows_per_sub = n_nodes // NS
    zrows = rows_per_sub // 16
    assert rows_per_sub % zrows == 0 and zrows <= CHUNK_E
    assert nsuper % 2 == 0
    bpc = BLK_PER_CHUNK

    cp = pltpu.CompilerParams()
    if "needs_layout_passes" in pltpu.CompilerParams.__dataclass_fields__:
        cp = dataclasses.replace(cp, needs_layout_passes=False)
    if "use_tc_tiling_on_sc" in pltpu.CompilerParams.__dataclass_fields__:
        cp = dataclasses.replace(cp, use_tc_tiling_on_sc=False)

    @functools.partial(
        pl.kernel,
        mesh=mesh,
        compiler_params=cp,
        out_type=jax.ShapeDtypeStruct((NC, n_nodes, L), jnp.float32),
        scratch_types=[
            pltpu.VMEM((SUP_ROWS, BLK), jnp.int32),
            pltpu.VMEM((SUP_ROWS, BLK), jnp.int32),
            pltpu.VMEM((CHUNK_E, L), jnp.float32),
            pltpu.VMEM((CHUNK_E, L), jnp.float32),
            pltpu.VMEM_SHARED((n_nodes, L), jnp.float32),
            pltpu.SemaphoreType.DMA,
            pltpu.SemaphoreType.DMA,
            pltpu.SemaphoreType.DMA,
            pltpu.SemaphoreType.DMA,
            pltpu.SemaphoreType.DMA,
            pltpu.SemaphoreType.DMA,
        ],
    )
    def seg_kernel(h_hbm, e_hbm, out_hbm,
                   ebuf0, ebuf1, rows0, rows1, acc,
                   esem0, esem1, gsem0, gsem1, ssem0, ssem1):
        c = lax.axis_index("c")
        s = lax.axis_index("s")
        wid = s * NC + c
        rowsb = (rows0, rows1)
        gsem = (gsem0, gsem1)
        ssem = (ssem0, ssem1)

        @pl.loop(0, zrows)
        def _zfill(i):
            rows0[i, :] = jnp.zeros((L,), jnp.float32)

        zcopies = [
            pltpu.async_copy(
                rows0.at[pl.ds(0, zrows)],
                acc.at[pl.ds(s * rows_per_sub + i * zrows, zrows)], esem0)
            for i in range(rows_per_sub // zrows)
        ]
        for z in zcopies:
            z.wait()

        plsc.subcore_barrier()

        sup0 = wid * nsuper

        def fire_gathers(ebuf, q, rows, sem):
            return [pltpu.async_copy(h_hbm.at[ebuf.at[12 * q + j]],
                                     rows.at[pl.ds(j * BLK, BLK)], sem)
                    for j in range(bpc)]

        def scale_rows(ebuf, q, rows):
            for j in range(bpc):
                wrow = 12 * q + 2 * bpc + j

                @pl.loop(0, BLK, step=16)
                def _scale(e, j=j, wrow=wrow):
                    wvec = plsc.bitcast(ebuf[wrow, pl.ds(e, L)], jnp.float32)
                    for k in range(L):
                        wb = lax.gather(
                            wvec, jnp.full((L, 1), k, jnp.int32),
                            lax.GatherDimensionNumbers(
                                offset_dims=(), collapsed_slice_dims=(0,),
                                start_index_map=(0,)),
                            (1,), mode=lax.GatherScatterMode.PROMISE_IN_BOUNDS)
                        idx = j * BLK + e + k
                        rows[idx, :] = rows[idx, :] * wb

        def fire_scatters(ebuf, q, rows, sem):
            return [pltpu.async_copy(rows.at[pl.ds(j * BLK, BLK)],
                                     acc.at[ebuf.at[12 * q + bpc + j]], sem,
                                     add=True)
                    for j in range(bpc)]

        def process4(ebuf):
            for qq in range(0, CPS, 2):
                g0 = fire_gathers(ebuf, qq, rows0, gsem0)
                g1 = fire_gathers(ebuf, qq + 1, rows1, gsem1)
                for h in g0:
                    h.wait()
                scale_rows(ebuf, qq, rows0)
                s0 = fire_scatters(ebuf, qq, rows0, ssem0)
                for h in g1:
                    h.wait()
                scale_rows(ebuf, qq + 1, rows1)
                s1 = fire_scatters(ebuf, qq + 1, rows1, ssem1)
                for h in s0 + s1:
                    h.wait()
            return []

        @pl.loop(0, nsuper, step=2)
        def _pair(t):
            pltpu.sync_copy(e_hbm.at[sup0 + t], ebuf0)
            pltpu.async_copy(e_hbm.at[sup0 + t + 1], ebuf1, esem1)
            process4(ebuf0)
            pltpu.make_async_copy(e_hbm.at[sup0 + t + 1], ebuf1, esem1).wait()
            process4(ebuf1)

        plsc.subcore_barrier()

        pltpu.sync_copy(acc.at[pl.ds(s * rows_per_sub, rows_per_sub)],
                        out_hbm.at[c].at[pl.ds(s * rows_per_sub, rows_per_sub)])

    return seg_kernel(h_pad, epacked)


def _norm_sq_max(hcat, n_nodes):
    br = 3128
    steps = n_nodes // br
    assert n_nodes % br == 0

    def body(h_ref, o_ref):
        i = pl.program_id(0)
        x = h_ref[...]
        m = jnp.max(jnp.sum(x * x, axis=1))

        @pl.when(i == 0)
        def _init():
            o_ref[0, 0] = m

        @pl.when(i > 0)
        def _acc():
            o_ref[0, 0] = jnp.maximum(o_ref[0, 0], m)

    return pl.pallas_call(
        body,
        grid=(steps,),
        in_specs=[pl.BlockSpec((br, L), lambda i: (i, 0))],
        out_specs=pl.BlockSpec(memory_space=pltpu.SMEM),
        out_shape=jax.ShapeDtypeStruct((1, 1), jnp.float32),
    )(hcat)


def _update_layer(h8, icat8, p08, p18, maxss, dt, wbd, btile,
                  scale_h, scale_agg, n8):
    def body(ms_ref, dt_ref, h_ref, i_ref, p0_ref, p1_ref, w_ref, b_ref, o_ref):
        s = lax.rsqrt(ms_ref[0, 0])
        dtv = dt_ref[0]
        h = h_ref[...]
        hs = h * s if scale_h else h
        agg = p0_ref[...] + p1_ref[...]
        a = jnp.dot(agg, w_ref[...], preferred_element_type=jnp.float32,
                    precision=lax.Precision.HIGHEST)
        if scale_agg:
            a = a * s
        t = jnp.tanh(a + b_ref[...])
        init_s = i_ref[...] * s
        o_ref[...] = hs + dtv * (t - hs + init_s)

    br = 3128
    assert n8 % br == 0
    return pl.pallas_call(
        body,
        grid=(n8 // br,),
        in_specs=[
            pl.BlockSpec(memory_space=pltpu.SMEM),
            pl.BlockSpec(memory_space=pltpu.SMEM),
            pl.BlockSpec((br, 128), lambda i: (i, 0)),
            pl.BlockSpec((br, 128), lambda i: (i, 0)),
            pl.BlockSpec((br, 128), lambda i: (i, 0)),
            pl.BlockSpec((br, 128), lambda i: (i, 0)),
            pl.BlockSpec((128, 128), lambda i: (0, 0)),
            pl.BlockSpec((1, 128), lambda i: (0, 0)),
        ],
        out_specs=pl.BlockSpec((br, 128), lambda i: (i, 0)),
        out_shape=jax.ShapeDtypeStruct((n8, 128), jnp.float32),
    )(maxss, dt, h8, icat8, p08, p18, wbd, btile)


def kernel(edge_index, edge_weight, dt, xu, xi, static_u, static_i,
           W0, b0, W1, b1):
    n_users, d = xu.shape
    n_items = xi.shape[0]
    n = n_users + n_items
    e = edge_weight.shape[0]
    npad = -(-n // 128) * 128
    n8 = npad * L // 128

    hcat = jnp.zeros((npad, L), jnp.float32)
    hcat = hcat.at[:n_users, :d].set(xu).at[n_users:n, :d].set(xi)
    icat = jnp.zeros((npad, L), jnp.float32)
    icat = icat.at[:n_users, :d].set(static_u).at[n_users:n, :d].set(static_i)

    nsuper = -(-e // (NW * SUP_E))
    nsuper += nsuper % 2
    e_pad = NW * SUP_E * nsuper
    pad = e_pad - e
    nsup_total = e_pad // SUP_E
    src_p = jnp.concatenate(
        [edge_index[0], jnp.zeros((pad,), jnp.int32)]
    ).reshape(nsup_total, CPS, BLK_PER_CHUNK, BLK)
    dst_p = jnp.concatenate(
        [edge_index[1], jnp.zeros((pad,), jnp.int32)]
    ).reshape(nsup_total, CPS, BLK_PER_CHUNK, BLK)
    w_p = jax.lax.bitcast_convert_type(
        jnp.concatenate([edge_weight, jnp.zeros((pad,), jnp.float32)]),
        jnp.int32).reshape(nsup_total, CPS, BLK_PER_CHUNK, BLK)
    epacked = jnp.concatenate([src_p, dst_p, w_p], axis=2).reshape(
        nsup_total, SUP_ROWS, BLK)

    def bdiag(w, b):
        wp = jnp.zeros((L, L), jnp.float32).at[:d, :d].set(w)
        bp = jnp.zeros((L,), jnp.float32).at[:d].set(b)
        return jnp.kron(jnp.eye(8, dtype=jnp.float32), wp), jnp.tile(bp, 8)[None, :]

    wbd0, bt0 = bdiag(W0, b0)
    wbd1, bt1 = bdiag(W1, b1)

    maxss = _norm_sq_max(hcat, npad)

    parts1 = _sc_weighted_segsum(hcat, epacked, npad, nsuper)
    p1a = parts1[0].reshape(n8, 128)
    p1b = parts1[1].reshape(n8, 128)
    h1_8 = _update_layer(hcat.reshape(n8, 128), icat.reshape(n8, 128),
                         p1a, p1b, maxss, dt, wbd0, bt0,
                         scale_h=True, scale_agg=True, n8=n8)

    h1 = h1_8.reshape(npad, L)
    parts2 = _sc_weighted_segsum(h1, epacked, npad, nsuper)
    p2a = parts2[0].reshape(n8, 128)
    p2b = parts2[1].reshape(n8, 128)
    h2_8 = _update_layer(h1_8, icat.reshape(n8, 128),
                         p2a, p2b, maxss, dt, wbd1, bt1,
                         scale_h=False, scale_agg=False, n8=n8)

    h2 = h2_8.reshape(npad, L)
    yu = h2[:n_users, :d]
    yi = h2[n_users:n, :d]
    return (yu, yi)
